# baseline (speedup 1.0000x reference)
import functools

import jax
import jax.numpy as jnp
from jax import lax
from jax.experimental import pallas as pl
from jax.experimental.pallas import tpu as pltpu
from jax.experimental.pallas import tpu_sc as plsc

_SC_PARAMS = pltpu.CompilerParams(use_tc_tiling_on_sc=False)

_NC = 2
_NS = 16
_CHUNK = 128


def _round_up(a, b):
    return (a + b - 1) // b * b


def _copy_out(sh_ref, out_ref, c, s, Z, L_last):
    @pl.when(s == _NS - 1)
    def _():
        pltpu.sync_copy(sh_ref.at[pl.ds((_NS - 1) * Z, L_last)],
                        out_ref.at[c, pl.ds((_NS - 1) * Z, L_last)])

    @pl.when(s != _NS - 1)
    def _():
        pltpu.sync_copy(sh_ref.at[pl.ds(s * Z, Z)],
                        out_ref.at[c, pl.ds(s * Z, Z)])


def _copy_in(hbm_ref, sh_ref, s, Z, L_last):
    @pl.when(s == _NS - 1)
    def _():
        pltpu.sync_copy(hbm_ref.at[pl.ds((_NS - 1) * Z, L_last)],
                        sh_ref.at[pl.ds((_NS - 1) * Z, L_last)])

    @pl.when(s != _NS - 1)
    def _():
        pltpu.sync_copy(hbm_ref.at[pl.ds(s * Z, Z)],
                        sh_ref.at[pl.ds(s * Z, Z)])


def _make_deg_kernel(n_groups, gw, n_nodes, Z, w):
    gpw = n_groups // (_NC * _NS)
    n_sp = Z * _NS
    l_last = n_nodes - (_NS - 1) * Z
    mesh = plsc.VectorSubcoreMesh(core_axis_name="c", subcore_axis_name="s")

    @functools.partial(
        pl.kernel,
        out_type=jax.ShapeDtypeStruct((_NC, n_nodes, w), jnp.float32),
        mesh=mesh,
        compiler_params=_SC_PARAMS,
        scratch_types=[
            pltpu.VMEM((gpw, gw), jnp.int32),
            pltpu.VMEM((gw, w), jnp.float32),
            pltpu.VMEM_SHARED((n_sp, w), jnp.float32),
        ],
    )
    def deg_kernel(dst_hbm, ones_hbm, zeros_hbm, out_hbm, dst_v, ones_v, deg_sh):
        c = lax.axis_index("c")
        s = lax.axis_index("s")
        wid = c * _NS + s
        pltpu.sync_copy(zeros_hbm, deg_sh.at[pl.ds(s * Z, Z)])
        pltpu.sync_copy(ones_hbm, ones_v)
        pltpu.sync_copy(dst_hbm.at[pl.ds(wid * gpw, gpw)], dst_v)
        plsc.subcore_barrier()

        def body(g, carry):
            pltpu.sync_copy(ones_v, deg_sh.at[dst_v.at[g]], add=True)
            return carry

        lax.fori_loop(0, gpw, body, 0)
        plsc.subcore_barrier()
        _copy_out(deg_sh, out_hbm, c, s, Z, l_last)

    return deg_kernel


def _make_agg_kernel(n_groups, gw, n_nodes, Z, h):
    gpw = n_groups // (_NC * _NS)
    n_sp = Z * _NS
    l_last = n_nodes - (_NS - 1) * Z
    mesh = plsc.VectorSubcoreMesh(core_axis_name="c", subcore_axis_name="s")

    @functools.partial(
        pl.kernel,
        out_type=jax.ShapeDtypeStruct((_NC, n_nodes, h), jnp.float32),
        mesh=mesh,
        compiler_params=_SC_PARAMS,
        scratch_types=[
            pltpu.VMEM((gpw, gw), jnp.int32),
            pltpu.VMEM((gpw, gw), jnp.int32),
            pltpu.VMEM((2, gw, h), jnp.float32),
            pltpu.VMEM_SHARED((n_sp, h), jnp.float32),
            pltpu.SemaphoreType.DMA,
        ],
    )
    def agg_kernel(src_hbm, dst_hbm, hp_hbm, zeros_hbm, out_hbm,
                   src_v, dst_v, rows_v, agg_sh, sem):
        c = lax.axis_index("c")
        s = lax.axis_index("s")
        pltpu.sync_copy(zeros_hbm, agg_sh.at[pl.ds(s * Z, Z)])
        plsc.subcore_barrier()

        @pl.when(c == 0)
        def _():
            for phase in range(_NC):
                w = phase * _NS + s
                pltpu.sync_copy(src_hbm.at[pl.ds(w * gpw, gpw)], src_v)
                pltpu.sync_copy(dst_hbm.at[pl.ds(w * gpw, gpw)], dst_v)

                pltpu.async_copy(hp_hbm.at[src_v.at[0]], rows_v.at[0], sem)

                def body(g, carry):
                    cur = lax.rem(g, 2)
                    nxt = 1 - cur

                    @pl.when(g < gpw - 1)
                    def _():
                        pltpu.async_copy(hp_hbm.at[src_v.at[g + 1]],
                                         rows_v.at[nxt], sem)

                    pltpu.make_async_copy(hp_hbm.at[src_v.at[0]],
                                          rows_v.at[cur], sem).wait()
                    pltpu.sync_copy(rows_v.at[cur], agg_sh.at[dst_v.at[g]],
                                    add=True)
                    return carry

                lax.fori_loop(0, gpw, body, 0)

        plsc.subcore_barrier()
        _copy_out(agg_sh, out_hbm, c, s, Z, l_last)

    return agg_kernel


def _xw_body(x_ref, w_ref, o_ref):
    o_ref[...] = jnp.dot(x_ref[...], w_ref[...],
                         preferred_element_type=jnp.float32)


def _prep_body(degp_ref, h_ref, dinv_ref, hp_ref):
    deg = degp_ref[0, :, 0:1] + degp_ref[1, :, 0:1] + 1.0
    dv = lax.rsqrt(deg)
    dinv_ref[...] = dv
    hp_ref[...] = h_ref[...] * dv


def _mid_body(p_ref, hp_ref, dinv_ref, b_ref, w_ref, o_ref):
    t = p_ref[0, :, :] + p_ref[1, :, :] + hp_ref[...]
    t = jnp.maximum(t * dinv_ref[...] + b_ref[...], 0.0)
    o_ref[...] = jnp.dot(t, w_ref[...],
                         preferred_element_type=jnp.float32) * dinv_ref[...]


def _fin_body(p_ref, hp_ref, dinv_ref, b_ref, wfc_ref, bfc_ref, o_ref):
    t = p_ref[0, :, :] + p_ref[1, :, :] + hp_ref[...]
    t = jnp.maximum(t * dinv_ref[...] + b_ref[...], 0.0)
    o_ref[...] = jnp.dot(t, wfc_ref[...],
                         preferred_element_type=jnp.float32) + bfc_ref[...]


def kernel(x, edge_index, batch, W1, b1, W2, b2, W3, b3, Wfc, bfc):
    n, d = x.shape
    h = W1.shape[1]
    o = Wfc.shape[1]
    e = edge_index.shape[1]
    f32 = jnp.float32

    gw = 4 * _CHUNK
    n_groups = _round_up(-(-e // gw), _NC * _NS * 2)
    e_pad = n_groups * gw
    z = _round_up(-(-(n + 1) // _NS), 8)

    src = jnp.concatenate(
        [edge_index[0], jnp.zeros((e_pad - e,), jnp.int32)]).reshape(n_groups, gw)
    dst = jnp.concatenate(
        [edge_index[1], jnp.full((e_pad - e,), n, jnp.int32)]).reshape(n_groups, gw)

    ones16 = jnp.ones((gw, 16), f32)
    zeros16 = jnp.zeros((z, 16), f32)
    zeros_h = jnp.zeros((z, h), f32)

    deg_k = _make_deg_kernel(n_groups, gw, n, z, 16)
    agg_k = _make_agg_kernel(n_groups, gw, n, z, h)

    sds = jax.ShapeDtypeStruct
    degp = deg_k(dst, ones16, zeros16)
    h1 = pl.pallas_call(_xw_body, out_shape=sds((n, h), f32))(x, W1)
    dinv, h1p = pl.pallas_call(
        _prep_body, out_shape=(sds((n, 1), f32), sds((n, h), f32)))(degp, h1)

    b1r = b1.reshape(1, h)
    b2r = b2.reshape(1, h)
    b3r = b3.reshape(1, h)
    bfcr = bfc.reshape(1, o)

    p1 = agg_k(src, dst, h1p, zeros_h)
    h2p = pl.pallas_call(_mid_body, out_shape=sds((n, h), f32))(
        p1, h1p, dinv, b1r, W2)
    p2 = agg_k(src, dst, h2p, zeros_h)
    h3p = pl.pallas_call(_mid_body, out_shape=sds((n, h), f32))(
        p2, h2p, dinv, b2r, W3)
    p3 = agg_k(src, dst, h3p, zeros_h)
    out = pl.pallas_call(_fin_body, out_shape=sds((n, o), f32))(
        p3, h3p, dinv, b3r, Wfc, bfcr)
    return out


# speedup vs baseline: 2.6390x; 2.6390x over previous
import functools

import jax
import jax.numpy as jnp
from jax import lax
from jax.experimental import pallas as pl
from jax.experimental.pallas import tpu as pltpu
from jax.experimental.pallas import tpu_sc as plsc

_SC_PARAMS = pltpu.CompilerParams(use_tc_tiling_on_sc=False)

_NC = 2
_NS = 16
_CHUNK = 128


def _round_up(a, b):
    return (a + b - 1) // b * b


def _copy_out(sh_ref, out_ref, c, s, Z, L_last):
    @pl.when(s == _NS - 1)
    def _():
        pltpu.sync_copy(sh_ref.at[pl.ds((_NS - 1) * Z, L_last)],
                        out_ref.at[c, pl.ds((_NS - 1) * Z, L_last)])

    @pl.when(s != _NS - 1)
    def _():
        pltpu.sync_copy(sh_ref.at[pl.ds(s * Z, Z)],
                        out_ref.at[c, pl.ds(s * Z, Z)])


def _copy_in(hbm_ref, sh_ref, s, Z, L_last):
    @pl.when(s == _NS - 1)
    def _():
        pltpu.sync_copy(hbm_ref.at[pl.ds((_NS - 1) * Z, L_last)],
                        sh_ref.at[pl.ds((_NS - 1) * Z, L_last)])

    @pl.when(s != _NS - 1)
    def _():
        pltpu.sync_copy(hbm_ref.at[pl.ds(s * Z, Z)],
                        sh_ref.at[pl.ds(s * Z, Z)])


def _make_deg_kernel(n_groups, gw, n_nodes, Z, w):
    gpw = n_groups // (_NC * _NS)
    n_sp = Z * _NS
    l_last = n_nodes - (_NS - 1) * Z
    mesh = plsc.VectorSubcoreMesh(core_axis_name="c", subcore_axis_name="s")

    @functools.partial(
        pl.kernel,
        out_type=jax.ShapeDtypeStruct((_NC, n_nodes, w), jnp.float32),
        mesh=mesh,
        compiler_params=_SC_PARAMS,
        scratch_types=[
            pltpu.VMEM((gpw, gw), jnp.int32),
            pltpu.VMEM((gw, w), jnp.float32),
            pltpu.VMEM_SHARED((n_sp, w), jnp.float32),
        ],
    )
    def deg_kernel(dst_hbm, ones_hbm, zeros_hbm, out_hbm, dst_v, ones_v, deg_sh):
        c = lax.axis_index("c")
        s = lax.axis_index("s")
        wid = c * _NS + s
        pltpu.sync_copy(zeros_hbm, deg_sh.at[pl.ds(s * Z, Z)])
        pltpu.sync_copy(ones_hbm, ones_v)
        pltpu.sync_copy(dst_hbm.at[pl.ds(wid * gpw, gpw)], dst_v)
        plsc.subcore_barrier()

        def body(g, carry):
            pltpu.sync_copy(ones_v, deg_sh.at[dst_v.at[g]], add=True)
            return carry

        lax.fori_loop(0, gpw, body, 0)
        plsc.subcore_barrier()
        _copy_out(deg_sh, out_hbm, c, s, Z, l_last)

    return deg_kernel


def _make_agg_kernel(n_groups, gw, n_nodes, Z, h):
    gpw = n_groups // (_NC * _NS)
    n_sp = Z * _NS
    l_last = n_nodes - (_NS - 1) * Z
    mesh = plsc.VectorSubcoreMesh(core_axis_name="c", subcore_axis_name="s")

    @functools.partial(
        pl.kernel,
        out_type=jax.ShapeDtypeStruct((_NC, n_nodes, h), jnp.float32),
        mesh=mesh,
        compiler_params=_SC_PARAMS,
        scratch_types=[
            pltpu.VMEM((gpw, gw), jnp.int32),
            pltpu.VMEM((gpw, gw), jnp.int32),
            pltpu.VMEM((2, gw, h), jnp.float32),
            pltpu.VMEM_SHARED((n_sp, h), jnp.float32),
            pltpu.SemaphoreType.DMA,
        ],
    )
    def agg_kernel(src_hbm, dst_hbm, hp_hbm, zeros_hbm, out_hbm,
                   src_v, dst_v, rows_v, agg_sh, sem):
        c = lax.axis_index("c")
        s = lax.axis_index("s")
        wid = c * _NS + s
        pltpu.sync_copy(zeros_hbm, agg_sh.at[pl.ds(s * Z, Z)])
        pltpu.sync_copy(src_hbm.at[pl.ds(wid * gpw, gpw)], src_v)
        pltpu.sync_copy(dst_hbm.at[pl.ds(wid * gpw, gpw)], dst_v)
        plsc.subcore_barrier()

        pltpu.async_copy(hp_hbm.at[src_v.at[0]], rows_v.at[0], sem)

        def body(g, carry):
            cur = lax.rem(g, 2)
            nxt = 1 - cur

            @pl.when(g < gpw - 1)
            def _():
                pltpu.async_copy(hp_hbm.at[src_v.at[g + 1]],
                                 rows_v.at[nxt], sem)

            pltpu.make_async_copy(hp_hbm.at[src_v.at[0]], rows_v.at[cur],
                                  sem).wait()
            pltpu.sync_copy(rows_v.at[cur], agg_sh.at[dst_v.at[g]],
                            add=True)
            return carry

        lax.fori_loop(0, gpw, body, 0)
        plsc.subcore_barrier()
        _copy_out(agg_sh, out_hbm, c, s, Z, l_last)

    return agg_kernel


def _xw_body(x_ref, w_ref, o_ref):
    o_ref[...] = jnp.dot(x_ref[...], w_ref[...],
                         preferred_element_type=jnp.float32)


def _prep_body(degp_ref, h_ref, dinv_ref, hp_ref):
    deg = degp_ref[0, :, 0:1] + degp_ref[1, :, 0:1] + 1.0
    dv = lax.rsqrt(deg)
    dinv_ref[...] = dv
    hp_ref[...] = h_ref[...] * dv


def _mid_body(p_ref, hp_ref, dinv_ref, b_ref, w_ref, o_ref):
    t = p_ref[0, :, :] + p_ref[1, :, :] + hp_ref[...]
    t = jnp.maximum(t * dinv_ref[...] + b_ref[...], 0.0)
    o_ref[...] = jnp.dot(t, w_ref[...],
                         preferred_element_type=jnp.float32) * dinv_ref[...]


def _fin_body(p_ref, hp_ref, dinv_ref, b_ref, wfc_ref, bfc_ref, o_ref):
    t = p_ref[0, :, :] + p_ref[1, :, :] + hp_ref[...]
    t = jnp.maximum(t * dinv_ref[...] + b_ref[...], 0.0)
    o_ref[...] = jnp.dot(t, wfc_ref[...],
                         preferred_element_type=jnp.float32) + bfc_ref[...]


def kernel(x, edge_index, batch, W1, b1, W2, b2, W3, b3, Wfc, bfc):
    n, d = x.shape
    h = W1.shape[1]
    o = Wfc.shape[1]
    e = edge_index.shape[1]
    f32 = jnp.float32

    gw = 4 * _CHUNK
    n_groups = _round_up(-(-e // gw), _NC * _NS * 2)
    e_pad = n_groups * gw
    z = _round_up(-(-(n + 1) // _NS), 8)

    n_sp = z * _NS
    pad_ar = jnp.arange(e_pad - e, dtype=jnp.int32)
    src = jnp.concatenate(
        [edge_index[0], pad_ar % n]).reshape(n_groups, gw)
    dst = jnp.concatenate(
        [edge_index[1], n + pad_ar % (n_sp - n)]).reshape(n_groups, gw)

    ones16 = jnp.ones((gw, 16), f32)
    zeros16 = jnp.zeros((z, 16), f32)
    zeros_h = jnp.zeros((z, h), f32)

    deg_k = _make_deg_kernel(n_groups, gw, n, z, 16)
    agg_k = _make_agg_kernel(n_groups, gw, n, z, h)

    sds = jax.ShapeDtypeStruct
    degp = deg_k(dst, ones16, zeros16)
    h1 = pl.pallas_call(_xw_body, out_shape=sds((n, h), f32))(x, W1)
    dinv, h1p = pl.pallas_call(
        _prep_body, out_shape=(sds((n, 1), f32), sds((n, h), f32)))(degp, h1)

    b1r = b1.reshape(1, h)
    b2r = b2.reshape(1, h)
    b3r = b3.reshape(1, h)
    bfcr = bfc.reshape(1, o)

    p1 = agg_k(src, dst, h1p, zeros_h)
    h2p = pl.pallas_call(_mid_body, out_shape=sds((n, h), f32))(
        p1, h1p, dinv, b1r, W2)
    p2 = agg_k(src, dst, h2p, zeros_h)
    h3p = pl.pallas_call(_mid_body, out_shape=sds((n, h), f32))(
        p2, h2p, dinv, b2r, W3)
    p3 = agg_k(src, dst, h3p, zeros_h)
    out = pl.pallas_call(_fin_body, out_shape=sds((n, o), f32))(
        p3, h3p, dinv, b3r, Wfc, bfcr)
    return out


# speedup vs baseline: 2.6437x; 1.0018x over previous
import functools

import jax
import jax.numpy as jnp
from jax import lax
from jax.experimental import pallas as pl
from jax.experimental.pallas import tpu as pltpu
from jax.experimental.pallas import tpu_sc as plsc

_SC_PARAMS = pltpu.CompilerParams(use_tc_tiling_on_sc=False)

_NC = 2
_NS = 16
_CHUNK = 128


def _round_up(a, b):
    return (a + b - 1) // b * b


def _copy_out(sh_ref, out_ref, c, s, Z, L_last):
    @pl.when(s == _NS - 1)
    def _():
        pltpu.sync_copy(sh_ref.at[pl.ds((_NS - 1) * Z, L_last)],
                        out_ref.at[c, pl.ds((_NS - 1) * Z, L_last)])

    @pl.when(s != _NS - 1)
    def _():
        pltpu.sync_copy(sh_ref.at[pl.ds(s * Z, Z)],
                        out_ref.at[c, pl.ds(s * Z, Z)])


def _copy_in(hbm_ref, sh_ref, s, Z, L_last):
    @pl.when(s == _NS - 1)
    def _():
        pltpu.sync_copy(hbm_ref.at[pl.ds((_NS - 1) * Z, L_last)],
                        sh_ref.at[pl.ds((_NS - 1) * Z, L_last)])

    @pl.when(s != _NS - 1)
    def _():
        pltpu.sync_copy(hbm_ref.at[pl.ds(s * Z, Z)],
                        sh_ref.at[pl.ds(s * Z, Z)])


def _make_deg_kernel(n_groups, gw, n_nodes, Z, w):
    gpw = n_groups // (_NC * _NS)
    n_sp = Z * _NS
    l_last = n_nodes - (_NS - 1) * Z
    mesh = plsc.VectorSubcoreMesh(core_axis_name="c", subcore_axis_name="s")

    @functools.partial(
        pl.kernel,
        out_type=jax.ShapeDtypeStruct((_NC, n_nodes, w), jnp.float32),
        mesh=mesh,
        compiler_params=_SC_PARAMS,
        scratch_types=[
            pltpu.VMEM((gpw, gw), jnp.int32),
            pltpu.VMEM((gw, w), jnp.float32),
            pltpu.VMEM_SHARED((n_sp, w), jnp.float32),
        ],
    )
    def deg_kernel(dst_hbm, ones_hbm, zeros_hbm, out_hbm, dst_v, ones_v, deg_sh):
        c = lax.axis_index("c")
        s = lax.axis_index("s")
        wid = c * _NS + s
        pltpu.sync_copy(zeros_hbm, deg_sh.at[pl.ds(s * Z, Z)])
        pltpu.sync_copy(ones_hbm, ones_v)
        pltpu.sync_copy(dst_hbm.at[pl.ds(wid * gpw, gpw)], dst_v)
        plsc.subcore_barrier()

        def body(g, carry):
            pltpu.sync_copy(ones_v, deg_sh.at[dst_v.at[g]], add=True)
            return carry

        lax.fori_loop(0, gpw, body, 0)
        plsc.subcore_barrier()
        _copy_out(deg_sh, out_hbm, c, s, Z, l_last)

    return deg_kernel


def _make_agg_kernel(n_groups, gw, n_nodes, Z, h):
    gpw = n_groups // (_NC * _NS)
    n_sp = Z * _NS
    l_last = n_nodes - (_NS - 1) * Z
    mesh = plsc.VectorSubcoreMesh(core_axis_name="c", subcore_axis_name="s")

    @functools.partial(
        pl.kernel,
        out_type=jax.ShapeDtypeStruct((_NC, n_nodes, h), jnp.float32),
        mesh=mesh,
        compiler_params=_SC_PARAMS,
        scratch_types=[
            pltpu.VMEM((gpw, gw), jnp.int32),
            pltpu.VMEM((gpw, gw), jnp.int32),
            pltpu.VMEM((2, gw, h), jnp.float32),
            pltpu.VMEM_SHARED((n_sp, h), jnp.float32),
            pltpu.SemaphoreType.DMA,
        ],
    )
    def agg_kernel(src_hbm, dst_hbm, hp_hbm, zeros_hbm, out_hbm,
                   src_v, dst_v, rows_v, agg_sh, sem):
        c = lax.axis_index("c")
        s = lax.axis_index("s")
        wid = c * _NS + s
        pltpu.sync_copy(zeros_hbm, agg_sh.at[pl.ds(s * Z, Z)])
        pltpu.sync_copy(src_hbm.at[pl.ds(wid * gpw, gpw)], src_v)
        pltpu.sync_copy(dst_hbm.at[pl.ds(wid * gpw, gpw)], dst_v)
        plsc.subcore_barrier()

        pltpu.async_copy(hp_hbm.at[src_v.at[0]], rows_v.at[0], sem)

        def body(g, carry):
            cur = lax.rem(g, 2)
            nxt = 1 - cur

            @pl.when(g < gpw - 1)
            def _():
                pltpu.async_copy(hp_hbm.at[src_v.at[g + 1]],
                                 rows_v.at[nxt], sem)

            pltpu.make_async_copy(hp_hbm.at[src_v.at[0]], rows_v.at[cur],
                                  sem).wait()
            pltpu.sync_copy(rows_v.at[cur], agg_sh.at[dst_v.at[g]],
                            add=True)
            return carry

        lax.fori_loop(0, gpw, body, 0)
        plsc.subcore_barrier()
        _copy_out(agg_sh, out_hbm, c, s, Z, l_last)

    return agg_kernel


def _prep_body(degp_ref, x_ref, w_ref, dinv_ref, hp_ref):
    deg = degp_ref[0, :, 0:1] + degp_ref[1, :, 0:1] + 1.0
    dv = lax.rsqrt(deg)
    dinv_ref[...] = dv
    h = jnp.dot(x_ref[...], w_ref[...], preferred_element_type=jnp.float32)
    hp_ref[...] = h * dv


def _mid_body(p_ref, hp_ref, dinv_ref, b_ref, w_ref, o_ref):
    t = p_ref[0, :, :] + p_ref[1, :, :] + hp_ref[...]
    t = jnp.maximum(t * dinv_ref[...] + b_ref[...], 0.0)
    o_ref[...] = jnp.dot(t, w_ref[...],
                         preferred_element_type=jnp.float32) * dinv_ref[...]


def _fin_body(p_ref, hp_ref, dinv_ref, b_ref, wfc_ref, bfc_ref, o_ref):
    t = p_ref[0, :, :] + p_ref[1, :, :] + hp_ref[...]
    t = jnp.maximum(t * dinv_ref[...] + b_ref[...], 0.0)
    o_ref[...] = jnp.dot(t, wfc_ref[...],
                         preferred_element_type=jnp.float32) + bfc_ref[...]


def kernel(x, edge_index, batch, W1, b1, W2, b2, W3, b3, Wfc, bfc):
    n, d = x.shape
    h = W1.shape[1]
    o = Wfc.shape[1]
    e = edge_index.shape[1]
    f32 = jnp.float32

    gw = 4 * _CHUNK
    n_groups = _round_up(-(-e // gw), _NC * _NS * 2)
    e_pad = n_groups * gw
    z = _round_up(-(-(n + 1) // _NS), 8)

    n_sp = z * _NS
    pad_ar = jnp.arange(e_pad - e, dtype=jnp.int32)
    src = jnp.concatenate(
        [edge_index[0], pad_ar % n]).reshape(n_groups, gw)
    dst = jnp.concatenate(
        [edge_index[1], n + pad_ar % (n_sp - n)]).reshape(n_groups, gw)

    ones16 = jnp.ones((gw, 16), f32)
    zeros16 = jnp.zeros((z, 16), f32)
    zeros_h = jnp.zeros((z, h), f32)

    deg_k = _make_deg_kernel(n_groups, gw, n, z, 16)
    agg_k = _make_agg_kernel(n_groups, gw, n, z, h)

    sds = jax.ShapeDtypeStruct
    degp = deg_k(dst, ones16, zeros16)
    dinv, h1p = pl.pallas_call(
        _prep_body, out_shape=(sds((n, 1), f32), sds((n, h), f32)))(degp, x, W1)

    b1r = b1.reshape(1, h)
    b2r = b2.reshape(1, h)
    b3r = b3.reshape(1, h)
    bfcr = bfc.reshape(1, o)

    p1 = agg_k(src, dst, h1p, zeros_h)
    h2p = pl.pallas_call(_mid_body, out_shape=sds((n, h), f32))(
        p1, h1p, dinv, b1r, W2)
    p2 = agg_k(src, dst, h2p, zeros_h)
    h3p = pl.pallas_call(_mid_body, out_shape=sds((n, h), f32))(
        p2, h2p, dinv, b2r, W3)
    p3 = agg_k(src, dst, h3p, zeros_h)
    out = pl.pallas_call(_fin_body, out_shape=sds((n, o), f32))(
        p3, h3p, dinv, b3r, Wfc, bfcr)
    return out
